# Initial kernel scaffold; baseline (speedup 1.0000x reference)
import jax
import jax.numpy as jnp
from jax.experimental import pallas as pl


def kernel(node_feats, edge_attrs, edge_feats, lengths, edge_index, W_scalar, W_up, W1, b1, W2, b2, W3, W_out):
    raise NotImplementedError("write your pallas kernel here")


# speedup vs baseline: 1.0940x; 1.0940x over previous
import functools

import jax
import jax.numpy as jnp
from jax.experimental import pallas as pl
from jax.experimental.pallas import tpu as pltpu

N = 10000
E = 320000
D = 128
AVG_NUM_NEIGHBORS = 32.0

NBLK = 1000
EBLK = 2000


def _precompute_body(nf_ref, wsc_ref, wup_ref, w1s_ref, w1r_ref,
                     ps_ref, pr_ref, u_ref):
    nf = nf_ref[...]
    ns = jnp.dot(nf, wsc_ref[...], preferred_element_type=jnp.float32)
    ps_ref[...] = jnp.dot(ns, w1s_ref[...], preferred_element_type=jnp.float32)
    pr_ref[...] = jnp.dot(ns, w1r_ref[...], preferred_element_type=jnp.float32)
    u_ref[...] = jnp.dot(nf, wup_ref[...], preferred_element_type=jnp.float32)


def _precompute(node_feats, W_scalar, W_up, W1s, W1r):
    grid = (N // NBLK,)
    blk = pl.BlockSpec((NBLK, D), lambda i: (i, 0))
    wblk = pl.BlockSpec((D, D), lambda i: (0, 0))
    return pl.pallas_call(
        _precompute_body,
        grid=grid,
        in_specs=[blk, wblk, wblk, wblk, wblk],
        out_specs=[blk, blk, blk],
        out_shape=[jax.ShapeDtypeStruct((N, D), jnp.float32)] * 3,
    )(node_feats, W_scalar, W_up, W1s, W1r)


def _mlp_body(g_ref, ef_ref, ea_ref, w1e_ref, b1_ref, w2_ref, b2_ref,
              w3_ref, w_ref):
    g = g_ref[...]
    et = jnp.dot(ef_ref[...], w1e_ref[...], preferred_element_type=jnp.float32)
    h = g + et + b1_ref[...]
    h = h * jax.nn.sigmoid(h)
    h = jnp.dot(h, w2_ref[...], preferred_element_type=jnp.float32) + b2_ref[...]
    h = h * jax.nn.sigmoid(h)
    tpw = jnp.dot(h, w3_ref[...], preferred_element_type=jnp.float32)
    w_ref[...] = tpw * ea_ref[...]


def _mlp(G, ef_ext, edge_attrs, W1e, b1, W2, b2, W3, e_pad):
    grid = (e_pad // EBLK,)
    eblk = pl.BlockSpec((EBLK, D), lambda i: (i, 0))
    return pl.pallas_call(
        _mlp_body,
        grid=grid,
        in_specs=[
            eblk,
            pl.BlockSpec((EBLK, 17), lambda i: (i, 0)),
            pl.BlockSpec((EBLK, 1), lambda i: (i, 0)),
            pl.BlockSpec((17, D), lambda i: (0, 0)),
            pl.BlockSpec((1, D), lambda i: (0, 0)),
            pl.BlockSpec((D, D), lambda i: (0, 0)),
            pl.BlockSpec((1, D), lambda i: (0, 0)),
            pl.BlockSpec((D, D), lambda i: (0, 0)),
        ],
        out_specs=eblk,
        out_shape=jax.ShapeDtypeStruct((e_pad, D), jnp.float32),
    )(G, ef_ext, edge_attrs, W1e, b1, W2, b2, W3)


def _final_body(m_ref, wout_ref, out_ref):
    m = m_ref[0] + m_ref[1]
    out_ref[...] = jnp.dot(m, wout_ref[...],
                           preferred_element_type=jnp.float32) * (1.0 / AVG_NUM_NEIGHBORS)


def _final(message_parts, W_out):
    grid = (N // NBLK,)
    return pl.pallas_call(
        _final_body,
        grid=grid,
        in_specs=[
            pl.BlockSpec((2, NBLK, D), lambda i: (0, i, 0)),
            pl.BlockSpec((D, D), lambda i: (0, 0)),
        ],
        out_specs=pl.BlockSpec((NBLK, D), lambda i: (i, 0)),
        out_shape=jax.ShapeDtypeStruct((N, D), jnp.float32),
    )(message_parts, W_out)


def kernel(node_feats, edge_attrs, edge_feats, lengths, edge_index,
           W_scalar, W_up, W1, b1, W2, b2, W3, W_out):
    sender = edge_index[0].astype(jnp.int32)
    receiver = edge_index[1].astype(jnp.int32)

    W1s = W1[:D]
    W1r = W1[D:2 * D]
    W1e = W1[2 * D:]

    P_s, P_r, U = _precompute(node_feats, W_scalar, W_up, W1s, W1r)

    G = jnp.take(P_s, sender, axis=0) + jnp.take(P_r, receiver, axis=0)
    U_g = jnp.take(U, sender, axis=0)

    ef_ext = jnp.concatenate([edge_feats, lengths], axis=1)

    w = _mlp(G, ef_ext, edge_attrs, W1e,
             b1.reshape(1, D), W2, b2.reshape(1, D), W3, E)
    mji = w * U_g

    m0 = jnp.zeros((N, D), jnp.float32).at[receiver].add(mji)
    message_parts = jnp.stack([m0, jnp.zeros((N, D), jnp.float32)])

    out = _final(message_parts, W_out)
    return out.reshape(N, D, 1)


# speedup vs baseline: 1.9593x; 1.7910x over previous
import functools

import jax
import jax.numpy as jnp
from jax import lax
from jax.experimental import pallas as pl
from jax.experimental.pallas import tpu as pltpu
from jax.experimental.pallas import tpu_sc as plsc

N = 10000
E = 320000
D = 128
AVG_NUM_NEIGHBORS = 32.0

NBLK = 1000
EBLK = 2048

_NC, _NS = 2, 16
NW = _NC * _NS
CH = 128
KCH = 80
EPW = KCH * CH
E_PAD = NW * EPW
ROWS2D = E_PAD // CH
N_PAD = 10240
RPS = N_PAD // _NS
ZROWS = 128

_sc_mesh = plsc.VectorSubcoreMesh(core_axis_name="c", subcore_axis_name="s")


def _precompute_body(nf_ref, wsc_ref, wup_ref, w1s_ref, w1r_ref,
                     ps_ref, pr_ref, u_ref):
    nf = nf_ref[...]
    ns = jnp.dot(nf, wsc_ref[...], preferred_element_type=jnp.float32)
    ps_ref[...] = jnp.dot(ns, w1s_ref[...], preferred_element_type=jnp.float32)
    pr_ref[...] = jnp.dot(ns, w1r_ref[...], preferred_element_type=jnp.float32)
    u_ref[...] = jnp.dot(nf, wup_ref[...], preferred_element_type=jnp.float32)


def _precompute(node_feats, W_scalar, W_up, W1s, W1r):
    blk = pl.BlockSpec((NBLK, D), lambda i: (i, 0))
    wblk = pl.BlockSpec((D, D), lambda i: (0, 0))
    return pl.pallas_call(
        _precompute_body,
        grid=(N // NBLK,),
        in_specs=[blk, wblk, wblk, wblk, wblk],
        out_specs=[blk, blk, blk],
        out_shape=[jax.ShapeDtypeStruct((N, D), jnp.float32)] * 3,
    )(node_feats, W_scalar, W_up, W1s, W1r)


@functools.partial(
    pl.kernel,
    out_type=[jax.ShapeDtypeStruct((E_PAD, D), jnp.float32),
              jax.ShapeDtypeStruct((E_PAD, D), jnp.float32)],
    mesh=_sc_mesh,
    scratch_types=[
        pltpu.VMEM((KCH, CH), jnp.int32),
        pltpu.VMEM((KCH, CH), jnp.int32),
        pltpu.VMEM((CH, D), jnp.float32),
        pltpu.VMEM((CH, D), jnp.float32),
        pltpu.SemaphoreType.DMA,
        pltpu.SemaphoreType.DMA,
    ],
)
def _gather_sc(ps_hbm, pr_hbm, s2d_hbm, r2d_hbm, gs_hbm, gr_hbm,
               sidx, ridx, rows_s, rows_r, sem_s, sem_r):
    wid = lax.axis_index("s") * _NC + lax.axis_index("c")
    krow = wid * KCH
    pltpu.sync_copy(s2d_hbm.at[pl.ds(krow, KCH)], sidx)
    pltpu.sync_copy(r2d_hbm.at[pl.ds(krow, KCH)], ridx)
    ebase = wid * EPW

    def body(k, _):
        cps = pltpu.async_copy(ps_hbm.at[sidx.at[k]], rows_s, sem_s)
        cpr = pltpu.async_copy(pr_hbm.at[ridx.at[k]], rows_r, sem_r)
        cps.wait()
        pltpu.sync_copy(rows_s, gs_hbm.at[pl.ds(ebase + k * CH, CH)])
        cpr.wait()
        pltpu.sync_copy(rows_r, gr_hbm.at[pl.ds(ebase + k * CH, CH)])
        return 0

    lax.fori_loop(0, KCH, body, 0)


KCH2 = KCH // 2


@functools.partial(
    pl.kernel,
    out_type=jax.ShapeDtypeStruct((_NC * N_PAD, D), jnp.float32),
    mesh=_sc_mesh,
    scratch_types=[
        pltpu.VMEM((KCH2, CH), jnp.int32),
        pltpu.VMEM((KCH2, CH), jnp.int32),
        pltpu.VMEM((CH, D), jnp.float32),
        pltpu.VMEM((CH, D), jnp.float32),
        pltpu.VMEM_SHARED((N_PAD, D), jnp.float32),
        pltpu.SemaphoreType.DMA,
        pltpu.SemaphoreType.DMA,
    ],
)
def _scatter_sc(w_hbm, u_hbm, s2d_hbm, r2d_hbm, out_hbm,
                sidx, ridx, wrows, urows, acc, sem_w, sem_u):
    cid = lax.axis_index("c")
    sid = lax.axis_index("s")
    wid = sid * _NC + cid

    zero16 = jnp.zeros((16,), jnp.float32)

    def zrow(i, _):
        for j in range(D // 16):
            wrows[i, pl.ds(j * 16, 16)] = zero16
        return 0

    lax.fori_loop(0, ZROWS, zrow, 0)

    def zcopy(t, _):
        pltpu.sync_copy(wrows, acc.at[pl.ds(sid * RPS + t * ZROWS, ZROWS)])
        return 0

    lax.fori_loop(0, RPS // ZROWS, zcopy, 0)
    plsc.subcore_barrier()

    def half(h, _):
        krow = wid * KCH + h * KCH2
        pltpu.sync_copy(s2d_hbm.at[pl.ds(krow, KCH2)], sidx)
        pltpu.sync_copy(r2d_hbm.at[pl.ds(krow, KCH2)], ridx)
        ebase = wid * EPW + h * KCH2 * CH

        def body(k, _):
            cpw = pltpu.async_copy(w_hbm.at[pl.ds(ebase + k * CH, CH)], wrows,
                                   sem_w)
            cpu_ = pltpu.async_copy(u_hbm.at[sidx.at[k]], urows, sem_u)
            cpw.wait()
            cpu_.wait()

            def mrow(i, _):
                for j in range(D // 16):
                    sl = pl.ds(j * 16, 16)
                    wrows[i, sl] = wrows[i, sl] * urows[i, sl]
                return 0

            lax.fori_loop(0, CH, mrow, 0)
            pltpu.sync_copy(wrows, acc.at[ridx.at[k]], add=True)
            return 0

        lax.fori_loop(0, KCH2, body, 0)
        return 0

    lax.fori_loop(0, 2, half, 0)
    plsc.subcore_barrier()

    def wout(t, _):
        rb = sid * RPS + t * ZROWS
        pltpu.sync_copy(acc.at[pl.ds(rb, ZROWS)],
                        out_hbm.at[pl.ds(cid * N_PAD + rb, ZROWS)])
        return 0

    lax.fori_loop(0, RPS // ZROWS, wout, 0)


def _mlp_body(gs_ref, gr_ref, ef_ref, ea_ref, w1e_ref, b1_ref, w2_ref, b2_ref,
              w3_ref, w_ref):
    g = gs_ref[...] + gr_ref[...]
    et = jnp.dot(ef_ref[...], w1e_ref[...], preferred_element_type=jnp.float32)
    h = g + et + b1_ref[...]
    h = h * jax.nn.sigmoid(h)
    h = jnp.dot(h, w2_ref[...], preferred_element_type=jnp.float32) + b2_ref[...]
    h = h * jax.nn.sigmoid(h)
    tpw = jnp.dot(h, w3_ref[...], preferred_element_type=jnp.float32)
    w_ref[...] = tpw * ea_ref[...]


def _mlp(Gs, Gr, ef_ext, edge_attrs, W1e, b1, W2, b2, W3):
    eblk = pl.BlockSpec((EBLK, D), lambda i: (i, 0))
    return pl.pallas_call(
        _mlp_body,
        grid=(E_PAD // EBLK,),
        in_specs=[
            eblk,
            eblk,
            pl.BlockSpec((EBLK, 17), lambda i: (i, 0)),
            pl.BlockSpec((EBLK, 1), lambda i: (i, 0)),
            pl.BlockSpec((17, D), lambda i: (0, 0)),
            pl.BlockSpec((1, D), lambda i: (0, 0)),
            pl.BlockSpec((D, D), lambda i: (0, 0)),
            pl.BlockSpec((1, D), lambda i: (0, 0)),
            pl.BlockSpec((D, D), lambda i: (0, 0)),
        ],
        out_specs=eblk,
        out_shape=jax.ShapeDtypeStruct((E_PAD, D), jnp.float32),
    )(Gs, Gr, ef_ext, edge_attrs, W1e, b1, W2, b2, W3)


def _final_body(m_ref, wout_ref, out_ref):
    m = m_ref[0] + m_ref[1]
    out_ref[...] = jnp.dot(m, wout_ref[...],
                           preferred_element_type=jnp.float32) * (1.0 / AVG_NUM_NEIGHBORS)


def _final(message_parts, W_out):
    return pl.pallas_call(
        _final_body,
        grid=(N // NBLK,),
        in_specs=[
            pl.BlockSpec((2, NBLK, D), lambda i: (0, i, 0)),
            pl.BlockSpec((D, D), lambda i: (0, 0)),
        ],
        out_specs=pl.BlockSpec((NBLK, D), lambda i: (i, 0)),
        out_shape=jax.ShapeDtypeStruct((N, D), jnp.float32),
    )(message_parts, W_out)


def kernel(node_feats, edge_attrs, edge_feats, lengths, edge_index,
           W_scalar, W_up, W1, b1, W2, b2, W3, W_out):
    sender = edge_index[0].astype(jnp.int32)
    receiver = edge_index[1].astype(jnp.int32)

    W1s = W1[:D]
    W1r = W1[D:2 * D]
    W1e = W1[2 * D:]

    P_s, P_r, U = _precompute(node_feats, W_scalar, W_up, W1s, W1r)

    pad = E_PAD - E
    s2d = jnp.pad(sender, (0, pad)).reshape(ROWS2D, CH)
    r2d = jnp.pad(receiver, (0, pad)).reshape(ROWS2D, CH)
    ef_ext = jnp.pad(jnp.concatenate([edge_feats, lengths], axis=1),
                     ((0, pad), (0, 0)))
    ea_pad = jnp.pad(edge_attrs, ((0, pad), (0, 0)))

    Gs, Gr = _gather_sc(P_s, P_r, s2d, r2d)

    w = _mlp(Gs, Gr, ef_ext, ea_pad, W1e,
             b1.reshape(1, D), W2, b2.reshape(1, D), W3)

    message_parts = _scatter_sc(w, U, s2d, r2d).reshape(_NC, N_PAD, D)[:, :N, :]

    out = _final(message_parts, W_out)
    return out.reshape(N, D, 1)


# speedup vs baseline: 2.0846x; 1.0640x over previous
import functools

import jax
import jax.numpy as jnp
from jax import lax
from jax.experimental import pallas as pl
from jax.experimental.pallas import tpu as pltpu
from jax.experimental.pallas import tpu_sc as plsc

N = 10000
E = 320000
D = 128
AVG_NUM_NEIGHBORS = 32.0

NBLK = 1000
EBLK = 2048

_NC, _NS = 2, 16
NW = _NC * _NS
CH = 128
KCH = 80
EPW = KCH * CH
E_PAD = NW * EPW
ROWS2D = E_PAD // CH
N_PAD = 10240
RPS = N_PAD // _NS
ZROWS = 128

_sc_mesh = plsc.VectorSubcoreMesh(core_axis_name="c", subcore_axis_name="s")


def _precompute_body(nf_ref, wsc_ref, wup_ref, w1s_ref, w1r_ref,
                     ps_ref, pr_ref, u_ref):
    nf = nf_ref[...]
    ns = jnp.dot(nf, wsc_ref[...], preferred_element_type=jnp.float32)
    ps_ref[...] = jnp.dot(ns, w1s_ref[...], preferred_element_type=jnp.float32)
    pr_ref[...] = jnp.dot(ns, w1r_ref[...], preferred_element_type=jnp.float32)
    u_ref[...] = jnp.dot(nf, wup_ref[...], preferred_element_type=jnp.float32)


def _precompute(node_feats, W_scalar, W_up, W1s, W1r):
    blk = pl.BlockSpec((NBLK, D), lambda i: (i, 0))
    wblk = pl.BlockSpec((D, D), lambda i: (0, 0))
    return pl.pallas_call(
        _precompute_body,
        grid=(N // NBLK,),
        in_specs=[blk, wblk, wblk, wblk, wblk],
        out_specs=[blk, blk, blk],
        out_shape=[jax.ShapeDtypeStruct((N, D), jnp.float32)] * 3,
    )(node_feats, W_scalar, W_up, W1s, W1r)


TG = KCH // 2


@functools.partial(
    pl.kernel,
    out_type=jax.ShapeDtypeStruct((E_PAD, D), jnp.float32),
    mesh=_sc_mesh,
    scratch_types=[
        pltpu.VMEM((KCH, CH), jnp.int32),
        pltpu.VMEM((KCH, CH), jnp.int32),
        pltpu.VMEM((CH, D), jnp.float32),
        pltpu.VMEM((CH, D), jnp.float32),
        pltpu.VMEM((CH, D), jnp.float32),
        pltpu.VMEM((CH, D), jnp.float32),
        pltpu.SemaphoreType.DMA,
        pltpu.SemaphoreType.DMA,
        pltpu.SemaphoreType.DMA,
        pltpu.SemaphoreType.DMA,
    ],
)
def _gather_sc(ps_hbm, pr_hbm, s2d_hbm, r2d_hbm, g_hbm,
               sidx, ridx, rs0, rr0, rs1, rr1,
               sem_g0, sem_g1, sem_wb0, sem_wb1):
    wid = lax.axis_index("s") * _NC + lax.axis_index("c")
    krow = wid * KCH
    pltpu.sync_copy(s2d_hbm.at[pl.ds(krow, KCH)], sidx)
    pltpu.sync_copy(r2d_hbm.at[pl.ds(krow, KCH)], ridx)
    ebase = wid * EPW

    slots = ((rs0, rr0, sem_g0, sem_wb0), (rs1, rr1, sem_g1, sem_wb1))

    def fire_g(c, slot):
        rs, rr, sg, _ = slots[slot]
        pltpu.async_copy(ps_hbm.at[sidx.at[c]], rs, sg)
        pltpu.async_copy(pr_hbm.at[ridx.at[c]], rr, sg)

    def wait_g(slot):
        rs, rr, sg, _ = slots[slot]
        pltpu.make_async_copy(ps_hbm.at[sidx.at[0]], rs, sg).wait()
        pltpu.make_async_copy(pr_hbm.at[ridx.at[0]], rr, sg).wait()

    def wait_wb(slot):
        rs, _, _, swb = slots[slot]
        pltpu.make_async_copy(rs, g_hbm.at[pl.ds(0, CH)], swb).wait()

    fire_g(0, 0)

    def group(t, _):
        for j in (0, 1):
            c = 2 * t + j
            rs, rr, sg, swb = slots[j]
            wait_g(j)
            if j == 0:
                @pl.when(t > 0)
                def _():
                    wait_wb(1)
                fire_g(c + 1, 1)
            else:
                wait_wb(0)

                @pl.when(t < TG - 1)
                def _():
                    fire_g(c + 1, 0)

            def add_row(i, _):
                for q in range(D // 16):
                    sl = pl.ds(q * 16, 16)
                    rs[i, sl] = rs[i, sl] + rr[i, sl]
                return 0

            lax.fori_loop(0, CH, add_row, 0)
            pltpu.async_copy(rs, g_hbm.at[pl.ds(ebase + c * CH, CH)], swb)
        return 0

    lax.fori_loop(0, TG, group, 0)
    wait_wb(1)


CH_S = 64
KCS = EPW // CH_S
NSEG = 4
SEG = KCS // NSEG
TG_S = SEG // 2
ROWS2DS = E_PAD // CH_S


@functools.partial(
    pl.kernel,
    out_type=jax.ShapeDtypeStruct((_NC * N_PAD, D), jnp.float32),
    mesh=_sc_mesh,
    scratch_types=[
        pltpu.VMEM((SEG, CH_S), jnp.int32),
        pltpu.VMEM((SEG, CH_S), jnp.int32),
        pltpu.VMEM((CH_S, D), jnp.float32),
        pltpu.VMEM((CH_S, D), jnp.float32),
        pltpu.VMEM((CH_S, D), jnp.float32),
        pltpu.VMEM((CH_S, D), jnp.float32),
        pltpu.VMEM_SHARED((N_PAD, D), jnp.float32),
        pltpu.SemaphoreType.DMA,
        pltpu.SemaphoreType.DMA,
        pltpu.SemaphoreType.DMA,
        pltpu.SemaphoreType.DMA,
    ],
)
def _scatter_sc(w_hbm, u_hbm, s2d_hbm, r2d_hbm, out_hbm,
                sidx, ridx, w0, u0, w1, u1, acc,
                sem_l0, sem_l1, sem_sc0, sem_sc1):
    cid = lax.axis_index("c")
    sid = lax.axis_index("s")
    wid = sid * _NC + cid

    zero16 = jnp.zeros((16,), jnp.float32)

    def zrow(i, _):
        for q in range(D // 16):
            w0[i, pl.ds(q * 16, 16)] = zero16
        return 0

    lax.fori_loop(0, CH_S, zrow, 0)

    def zcopy(t, _):
        pltpu.sync_copy(w0, acc.at[pl.ds(sid * RPS + t * CH_S, CH_S)])
        return 0

    lax.fori_loop(0, RPS // CH_S, zcopy, 0)
    plsc.subcore_barrier()

    slots = ((w0, u0, sem_l0, sem_sc0), (w1, u1, sem_l1, sem_sc1))

    def half(h, _):
        krow = wid * KCS + h * SEG
        pltpu.sync_copy(s2d_hbm.at[pl.ds(krow, SEG)], sidx)
        pltpu.sync_copy(r2d_hbm.at[pl.ds(krow, SEG)], ridx)
        ebase = wid * EPW + h * SEG * CH_S

        def fire_l(c, slot):
            w, u, sl_, _ = slots[slot]
            pltpu.async_copy(w_hbm.at[pl.ds(ebase + c * CH_S, CH_S)], w, sl_)
            pltpu.async_copy(u_hbm.at[sidx.at[c]], u, sl_)

        def wait_l(slot):
            w, u, sl_, _ = slots[slot]
            pltpu.make_async_copy(w_hbm.at[pl.ds(0, CH_S)], w, sl_).wait()
            pltpu.make_async_copy(u_hbm.at[sidx.at[0]], u, sl_).wait()

        def wait_sc(slot):
            w, _, _, ssc = slots[slot]
            pltpu.make_async_copy(w, acc.at[ridx.at[0]], ssc).wait()

        fire_l(0, 0)

        def group(t, _):
            for j in (0, 1):
                c = 2 * t + j
                w, u, _, ssc = slots[j]
                wait_l(j)
                if j == 0:
                    @pl.when(t > 0)
                    def _():
                        wait_sc(1)

                    fire_l(c + 1, 1)
                else:
                    wait_sc(0)

                    @pl.when(t < TG_S - 1)
                    def _():
                        fire_l(c + 1, 0)

                def mrow(i, _):
                    for q in range(D // 16):
                        sl = pl.ds(q * 16, 16)
                        w[i, sl] = w[i, sl] * u[i, sl]
                    return 0

                lax.fori_loop(0, CH_S, mrow, 0)
                pltpu.async_copy(w, acc.at[ridx.at[c]], ssc, add=True)
            return 0

        lax.fori_loop(0, TG_S, group, 0)
        wait_sc(1)
        return 0

    lax.fori_loop(0, NSEG, half, 0)
    plsc.subcore_barrier()

    def wout(t, _):
        rb = sid * RPS + t * CH_S
        pltpu.sync_copy(acc.at[pl.ds(rb, CH_S)],
                        out_hbm.at[pl.ds(cid * N_PAD + rb, CH_S)])
        return 0

    lax.fori_loop(0, RPS // CH_S, wout, 0)


def _mlp_body(g_ref, ef_ref, ea_ref, w1e_ref, b1_ref, w2_ref, b2_ref,
              w3_ref, w_ref):
    g = g_ref[...]
    et = jnp.dot(ef_ref[...], w1e_ref[...], preferred_element_type=jnp.float32)
    h = g + et + b1_ref[...]
    h = h * jax.nn.sigmoid(h)
    h = jnp.dot(h, w2_ref[...], preferred_element_type=jnp.float32) + b2_ref[...]
    h = h * jax.nn.sigmoid(h)
    tpw = jnp.dot(h, w3_ref[...], preferred_element_type=jnp.float32)
    w_ref[...] = tpw * ea_ref[...]


def _mlp(G, ef_ext, edge_attrs, W1e, b1, W2, b2, W3):
    eblk = pl.BlockSpec((EBLK, D), lambda i: (i, 0))
    return pl.pallas_call(
        _mlp_body,
        grid=(E_PAD // EBLK,),
        in_specs=[
            eblk,
            pl.BlockSpec((EBLK, 17), lambda i: (i, 0)),
            pl.BlockSpec((EBLK, 1), lambda i: (i, 0)),
            pl.BlockSpec((17, D), lambda i: (0, 0)),
            pl.BlockSpec((1, D), lambda i: (0, 0)),
            pl.BlockSpec((D, D), lambda i: (0, 0)),
            pl.BlockSpec((1, D), lambda i: (0, 0)),
            pl.BlockSpec((D, D), lambda i: (0, 0)),
        ],
        out_specs=eblk,
        out_shape=jax.ShapeDtypeStruct((E_PAD, D), jnp.float32),
    )(G, ef_ext, edge_attrs, W1e, b1, W2, b2, W3)


def _final_body(m_ref, wout_ref, out_ref):
    m = m_ref[0] + m_ref[1]
    out_ref[...] = jnp.dot(m, wout_ref[...],
                           preferred_element_type=jnp.float32) * (1.0 / AVG_NUM_NEIGHBORS)


def _final(message_parts, W_out):
    return pl.pallas_call(
        _final_body,
        grid=(N // NBLK,),
        in_specs=[
            pl.BlockSpec((2, NBLK, D), lambda i: (0, i, 0)),
            pl.BlockSpec((D, D), lambda i: (0, 0)),
        ],
        out_specs=pl.BlockSpec((NBLK, D), lambda i: (i, 0)),
        out_shape=jax.ShapeDtypeStruct((N, D), jnp.float32),
    )(message_parts, W_out)


def kernel(node_feats, edge_attrs, edge_feats, lengths, edge_index,
           W_scalar, W_up, W1, b1, W2, b2, W3, W_out):
    sender = edge_index[0].astype(jnp.int32)
    receiver = edge_index[1].astype(jnp.int32)

    W1s = W1[:D]
    W1r = W1[D:2 * D]
    W1e = W1[2 * D:]

    P_s, P_r, U = _precompute(node_feats, W_scalar, W_up, W1s, W1r)

    pad = E_PAD - E
    s2d = jnp.pad(sender, (0, pad)).reshape(ROWS2D, CH)
    r2d = jnp.pad(receiver, (0, pad)).reshape(ROWS2D, CH)
    ef_ext = jnp.pad(jnp.concatenate([edge_feats, lengths], axis=1),
                     ((0, pad), (0, 0)))
    ea_pad = jnp.pad(edge_attrs, ((0, pad), (0, 0)))

    s2ds = s2d.reshape(ROWS2DS, CH_S)
    r2ds = r2d.reshape(ROWS2DS, CH_S)

    G = _gather_sc(P_s, P_r, s2d, r2d)

    w = _mlp(G, ef_ext, ea_pad, W1e,
             b1.reshape(1, D), W2, b2.reshape(1, D), W3)

    message_parts = _scatter_sc(w, U, s2ds, r2ds).reshape(_NC, N_PAD, D)[:, :N, :]

    out = _final(message_parts, W_out)
    return out.reshape(N, D, 1)
